# baseline (speedup 1.0000x reference)
import functools

import jax
import jax.numpy as jnp
from jax import lax
from jax.experimental import pallas as pl
from jax.experimental.pallas import tpu as pltpu
from jax.experimental.pallas import tpu_sc as plsc

N = 10000
D = 128
E = 320000
G = 32
W0 = D + 16

NP = 10112
ROWS_PER_TILE = NP // 16
NW = 32
CH = 128
MB = 8
NMACRO = -(-E // (NW * CH * MB))
NCHUNK = NMACRO * MB
EP = NW * CH * NCHUNK
PER_W = EP // NW
DUMMY = N


def _sc_segsum_body(width, table, src, dst, zrows, out, acc,
                    src_idx, dst_idx, rows0, rows1,
                    gsem0, gsem1, ssem0, ssem1):
    c = lax.axis_index("c")
    s = lax.axis_index("s")
    wid = s * 2 + c

    rows = (rows0, rows1)
    gsem = (gsem0, gsem1)
    ssem = (ssem0, ssem1)

    chunks = [(0, 128), (128, 128), (256, 128), (384, 128), (512, 120)]

    row0 = s * ROWS_PER_TILE
    pltpu.sync_copy(zrows, rows0)
    for off_r, cnt in chunks:
        pltpu.sync_copy(rows0.at[pl.ds(0, cnt)],
                        acc.at[pl.ds(row0 + off_r, cnt)])
    plsc.subcore_barrier()

    base = wid * NCHUNK

    def macro(t, _):
        crow = base + t * MB
        pltpu.sync_copy(src.at[pl.ds(crow, MB)], src_idx)
        pltpu.sync_copy(dst.at[pl.ds(crow, MB)], dst_idx)
        for m in range(MB):
            pltpu.async_copy(table.at[src_idx.at[m]], rows0, gsem0).wait()
            pltpu.sync_copy(rows0, acc.at[dst_idx.at[m]], add=True)
        return 0

    lax.fori_loop(0, NMACRO, macro, 0)
    plsc.subcore_barrier()

    out_row = c * NP + row0
    for off_r, cnt in chunks:
        pltpu.sync_copy(acc.at[pl.ds(row0 + off_r, cnt)],
                        rows0.at[pl.ds(0, cnt)])
        pltpu.sync_copy(rows0.at[pl.ds(0, cnt)],
                        out.at[pl.ds(out_row + off_r, cnt)])


@functools.lru_cache(maxsize=None)
def _sc_segsum_cached(width):
    mesh = plsc.VectorSubcoreMesh(core_axis_name="c", subcore_axis_name="s")
    return pl.kernel(
        functools.partial(_sc_segsum_body, width),
        out_type=jax.ShapeDtypeStruct((2 * NP, width), jnp.float32),
        mesh=mesh,
        scratch_types=[
            pltpu.VMEM_SHARED((NP, width), jnp.float32),
            pltpu.VMEM((MB, CH), jnp.int32),
            pltpu.VMEM((MB, CH), jnp.int32),
            pltpu.VMEM((CH, width), jnp.float32),
            pltpu.VMEM((CH, width), jnp.float32),
            pltpu.SemaphoreType.DMA,
            pltpu.SemaphoreType.DMA,
            pltpu.SemaphoreType.DMA,
            pltpu.SemaphoreType.DMA,
        ],
        compiler_params=pltpu.CompilerParams(use_tc_tiling_on_sc=False),
    )


def _segsum(table, src, dst, zrows):
    return _sc_segsum_cached(table.shape[1])(table, src, dst, zrows)


BN = 2528
NBLK = NP // BN
_HI = lax.Precision.HIGHEST


def _dense0_body(x, wn, wr, b, y, r):
    xb = x[...]
    y0 = jnp.dot(xb, wn[...], preferred_element_type=jnp.float32, precision=_HI)
    y[...] = jnp.concatenate([y0, jnp.ones((BN, 16), jnp.float32)], axis=1)
    r[...] = jnp.dot(xb, wr[...], preferred_element_type=jnp.float32,
                     precision=_HI) + b[...]


def _dense0(x, wn, wr, b):
    return pl.pallas_call(
        _dense0_body,
        grid=(NBLK,),
        in_specs=[
            pl.BlockSpec((BN, D), lambda i: (i, 0)),
            pl.BlockSpec((D, D), lambda i: (0, 0)),
            pl.BlockSpec((D, D), lambda i: (0, 0)),
            pl.BlockSpec((1, D), lambda i: (0, 0)),
        ],
        out_specs=[
            pl.BlockSpec((BN, W0), lambda i: (i, 0)),
            pl.BlockSpec((BN, D), lambda i: (i, 0)),
        ],
        out_shape=[
            jax.ShapeDtypeStruct((NP, W0), jnp.float32),
            jax.ShapeDtypeStruct((NP, D), jnp.float32),
        ],
    )(x, wn, wr, b)


def _norm_layer(msg, dg, r, g, be):
    h = msg / dg + r[...]
    mu = jnp.mean(h, axis=-1, keepdims=True)
    var = jnp.mean((h - mu) * (h - mu), axis=-1, keepdims=True)
    hn = (h - mu) * lax.rsqrt(var + 1e-5) * g[...] + be[...]
    return jnp.maximum(hn, 0.0)


def _fuse_mid_body(aggA, aggB, r0, g0, be0, wn, wr, b, y1, r1, dg8):
    S = aggA[...] + aggB[...]
    dg = jnp.maximum(S[:, D:D + 1], 1.0)
    h1 = _norm_layer(S[:, :D], dg, r0, g0, be0)
    y1[...] = jnp.dot(h1, wn[...], preferred_element_type=jnp.float32,
                      precision=_HI)
    r1[...] = jnp.dot(h1, wr[...], preferred_element_type=jnp.float32,
                      precision=_HI) + b[...]
    dg8[...] = lax.broadcast_in_dim(dg, (BN, 8), (0, 1))


def _fuse_mid(agg2, r0, g0, be0, wn, wr, b):
    return pl.pallas_call(
        _fuse_mid_body,
        grid=(NBLK,),
        in_specs=[
            pl.BlockSpec((BN, W0), lambda i: (i, 0)),
            pl.BlockSpec((BN, W0), lambda i: (i + NBLK, 0)),
            pl.BlockSpec((BN, D), lambda i: (i, 0)),
            pl.BlockSpec((1, D), lambda i: (0, 0)),
            pl.BlockSpec((1, D), lambda i: (0, 0)),
            pl.BlockSpec((D, D), lambda i: (0, 0)),
            pl.BlockSpec((D, D), lambda i: (0, 0)),
            pl.BlockSpec((1, D), lambda i: (0, 0)),
        ],
        out_specs=[
            pl.BlockSpec((BN, D), lambda i: (i, 0)),
            pl.BlockSpec((BN, D), lambda i: (i, 0)),
            pl.BlockSpec((BN, 8), lambda i: (i, 0)),
        ],
        out_shape=[
            jax.ShapeDtypeStruct((NP, D), jnp.float32),
            jax.ShapeDtypeStruct((NP, D), jnp.float32),
            jax.ShapeDtypeStruct((NP, 8), jnp.float32),
        ],
    )(agg2, agg2, r0, g0, be0, wn, wr, b)


def _fuse_out_body(aggA, aggB, dg8, r1, g1, be1, batch, wc, bc, out,
                   p_acc, c_acc):
    i = pl.program_id(0)

    @pl.when(i == 0)
    def _():
        p_acc[...] = jnp.zeros_like(p_acc)
        c_acc[...] = jnp.zeros_like(c_acc)

    S = aggA[...] + aggB[...]
    h2 = _norm_layer(S, dg8[:, 0:1], r1, g1, be1)

    gids = lax.broadcasted_iota(jnp.int32, (BN, G), 1)
    m = (batch[...] == gids).astype(jnp.float32)
    p_acc[...] += lax.dot_general(m, h2, (((0,), (0,)), ((), ())),
                                  preferred_element_type=jnp.float32,
                                  precision=_HI)
    c_acc[...] += lax.dot_general(m, jnp.ones((BN, D), jnp.float32),
                                  (((0,), (0,)), ((), ())),
                                  preferred_element_type=jnp.float32,
                                  precision=_HI)

    @pl.when(i == NBLK - 1)
    def _():
        pooled = p_acc[...] / jnp.maximum(c_acc[...], 1.0)
        out[...] = jnp.dot(pooled, wc[...], preferred_element_type=jnp.float32,
                           precision=_HI) + bc[...]


def _fuse_out(agg2, dg8, r1, g1, be1, batch, wc, bc):
    return pl.pallas_call(
        _fuse_out_body,
        grid=(NBLK,),
        in_specs=[
            pl.BlockSpec((BN, D), lambda i: (i, 0)),
            pl.BlockSpec((BN, D), lambda i: (i + NBLK, 0)),
            pl.BlockSpec((BN, 8), lambda i: (i, 0)),
            pl.BlockSpec((BN, D), lambda i: (i, 0)),
            pl.BlockSpec((1, D), lambda i: (0, 0)),
            pl.BlockSpec((1, D), lambda i: (0, 0)),
            pl.BlockSpec((BN, 1), lambda i: (i, 0)),
            pl.BlockSpec((D, D), lambda i: (0, 0)),
            pl.BlockSpec((1, D), lambda i: (0, 0)),
        ],
        out_specs=pl.BlockSpec((G, D), lambda i: (0, 0)),
        out_shape=jax.ShapeDtypeStruct((G, D), jnp.float32),
        scratch_shapes=[
            pltpu.VMEM((G, D), jnp.float32),
            pltpu.VMEM((G, D), jnp.float32),
        ],
    )(agg2, agg2, dg8, r1, g1, be1, batch, wc, bc)


@jax.jit
def kernel(x, edge_index, batch, W_nbr0, W_root0, b0, g0, be0,
           W_nbr1, W_root1, b1, g1, be1, Wc, bc):
    f32 = jnp.float32
    x_pad = jnp.zeros((NP, D), f32).at[:N].set(x)
    src = jnp.full((EP,), DUMMY, jnp.int32).at[:E].set(edge_index[0])
    src = src.reshape(EP // CH, CH)
    dst = jnp.full((EP,), DUMMY, jnp.int32).at[:E].set(edge_index[1])
    dst = dst.reshape(EP // CH, CH)
    batch_pad = jnp.full((NP, 1), G, jnp.int32).at[:N, 0].set(batch)

    zrows0 = jnp.zeros((CH, W0), f32)
    zrows1 = jnp.zeros((CH, D), f32)

    b0r = b0.reshape(1, D)
    b1r = b1.reshape(1, D)
    g0r = g0.reshape(1, D)
    be0r = be0.reshape(1, D)
    g1r = g1.reshape(1, D)
    be1r = be1.reshape(1, D)
    bcr = bc.reshape(1, D)

    y0, r0 = _dense0(x_pad, W_nbr0, W_root0, b0r)
    agg0 = _segsum(y0, src, dst, zrows0)
    y1, r1, dg8 = _fuse_mid(agg0, r0, g0r, be0r, W_nbr1, W_root1, b1r)
    agg1 = _segsum(y1, src, dst, zrows1)
    out = _fuse_out(agg1, dg8, r1, g1r, be1r, batch_pad, Wc, bcr)
    return out


# speedup vs baseline: 2.5006x; 2.5006x over previous
import functools

import jax
import jax.numpy as jnp
from jax import lax
from jax.experimental import pallas as pl
from jax.experimental.pallas import tpu as pltpu
from jax.experimental.pallas import tpu_sc as plsc

N = 10000
D = 128
E = 320000
G = 32
W0 = D + 16

NP = 10112
ROWS_PER_TILE = NP // 16
NW = 32
CH = 128
MB = 8
NMACRO = -(-E // (NW * CH * MB))
NCHUNK = NMACRO * MB
EP = NW * CH * NCHUNK
PER_W = EP // NW
DUMMY = N


def _sc_segsum_body(width, table, src, dst, zrows, out, acc,
                    src_idx, dst_idx, rows0, rows1,
                    gsem0, gsem1, ssem0, ssem1):
    c = lax.axis_index("c")
    s = lax.axis_index("s")
    wid = s * 2 + c

    rows = (rows0, rows1)
    gsem = (gsem0, gsem1)
    ssem = (ssem0, ssem1)

    chunks = [(0, 128), (128, 128), (256, 128), (384, 128), (512, 120)]

    row0 = s * ROWS_PER_TILE
    pltpu.sync_copy(zrows, rows0)
    for off_r, cnt in chunks:
        pltpu.sync_copy(rows0.at[pl.ds(0, cnt)],
                        acc.at[pl.ds(row0 + off_r, cnt)])
    plsc.subcore_barrier()

    base = wid * NCHUNK

    def macro(t, _):
        crow = base + t * MB
        pltpu.sync_copy(src.at[pl.ds(crow, MB)], src_idx)
        pltpu.sync_copy(dst.at[pl.ds(crow, MB)], dst_idx)
        for m in range(MB):
            pltpu.async_copy(table.at[src_idx.at[m]], rows0, gsem0).wait()
            pltpu.sync_copy(rows0, acc.at[dst_idx.at[m]], add=True)
        return 0

    lax.fori_loop(0, NMACRO, macro, 0)
    plsc.subcore_barrier()

    out_row = c * NP + row0
    for off_r, cnt in chunks:
        pltpu.sync_copy(acc.at[pl.ds(row0 + off_r, cnt)],
                        rows0.at[pl.ds(0, cnt)])
        pltpu.sync_copy(rows0.at[pl.ds(0, cnt)],
                        out.at[pl.ds(out_row + off_r, cnt)])


@functools.lru_cache(maxsize=None)
def _sc_segsum_cached(width):
    mesh = plsc.VectorSubcoreMesh(core_axis_name="c", subcore_axis_name="s")
    return pl.kernel(
        functools.partial(_sc_segsum_body, width),
        out_type=jax.ShapeDtypeStruct((2 * NP, width), jnp.float32),
        mesh=mesh,
        scratch_types=[
            pltpu.VMEM_SHARED((NP, width), jnp.float32),
            pltpu.VMEM((MB, CH), jnp.int32),
            pltpu.VMEM((MB, CH), jnp.int32),
            pltpu.VMEM((CH, width), jnp.float32),
            pltpu.VMEM((CH, width), jnp.float32),
            pltpu.SemaphoreType.DMA,
            pltpu.SemaphoreType.DMA,
            pltpu.SemaphoreType.DMA,
            pltpu.SemaphoreType.DMA,
        ],
        compiler_params=pltpu.CompilerParams(use_tc_tiling_on_sc=False),
    )


def _segsum(table, src, dst, zrows):
    return _sc_segsum_cached(table.shape[1])(table, src, dst, zrows)


BN = 2528
NBLK = NP // BN
_HI = lax.Precision.HIGHEST


def _dense0_body(x, wn, wr, b, y, r):
    xb = x[...]
    y0 = jnp.dot(xb, wn[...], preferred_element_type=jnp.float32, precision=_HI)
    y[...] = jnp.concatenate([y0, jnp.ones((BN, 16), jnp.float32)], axis=1)
    r[...] = jnp.dot(xb, wr[...], preferred_element_type=jnp.float32,
                     precision=_HI) + b[...]


def _dense0(x, wn, wr, b):
    return pl.pallas_call(
        _dense0_body,
        grid=(NBLK,),
        in_specs=[
            pl.BlockSpec((BN, D), lambda i: (i, 0)),
            pl.BlockSpec((D, D), lambda i: (0, 0)),
            pl.BlockSpec((D, D), lambda i: (0, 0)),
            pl.BlockSpec((1, D), lambda i: (0, 0)),
        ],
        out_specs=[
            pl.BlockSpec((BN, W0), lambda i: (i, 0)),
            pl.BlockSpec((BN, D), lambda i: (i, 0)),
        ],
        out_shape=[
            jax.ShapeDtypeStruct((NP, W0), jnp.float32),
            jax.ShapeDtypeStruct((NP, D), jnp.float32),
        ],
    )(x, wn, wr, b)


def _norm_layer(msg, dg, r, g, be):
    h = msg / dg + r[...]
    mu = jnp.mean(h, axis=-1, keepdims=True)
    var = jnp.mean((h - mu) * (h - mu), axis=-1, keepdims=True)
    hn = (h - mu) * lax.rsqrt(var + 1e-5) * g[...] + be[...]
    return jnp.maximum(hn, 0.0)


def _fuse_mid_body(aggA, aggB, r0, g0, be0, wn, wr, b, y1, r1, dg8):
    S = aggA[...] + aggB[...]
    dg = jnp.maximum(S[:, D:D + 1], 1.0)
    h1 = _norm_layer(S[:, :D], dg, r0, g0, be0)
    y1[...] = jnp.dot(h1, wn[...], preferred_element_type=jnp.float32,
                      precision=_HI)
    r1[...] = jnp.dot(h1, wr[...], preferred_element_type=jnp.float32,
                      precision=_HI) + b[...]
    dg8[...] = lax.broadcast_in_dim(dg, (BN, 8), (0, 1))


def _fuse_mid(agg2, r0, g0, be0, wn, wr, b):
    return pl.pallas_call(
        _fuse_mid_body,
        grid=(NBLK,),
        in_specs=[
            pl.BlockSpec((BN, W0), lambda i: (i, 0)),
            pl.BlockSpec((BN, W0), lambda i: (i + NBLK, 0)),
            pl.BlockSpec((BN, D), lambda i: (i, 0)),
            pl.BlockSpec((1, D), lambda i: (0, 0)),
            pl.BlockSpec((1, D), lambda i: (0, 0)),
            pl.BlockSpec((D, D), lambda i: (0, 0)),
            pl.BlockSpec((D, D), lambda i: (0, 0)),
            pl.BlockSpec((1, D), lambda i: (0, 0)),
        ],
        out_specs=[
            pl.BlockSpec((BN, D), lambda i: (i, 0)),
            pl.BlockSpec((BN, D), lambda i: (i, 0)),
            pl.BlockSpec((BN, 8), lambda i: (i, 0)),
        ],
        out_shape=[
            jax.ShapeDtypeStruct((NP, D), jnp.float32),
            jax.ShapeDtypeStruct((NP, D), jnp.float32),
            jax.ShapeDtypeStruct((NP, 8), jnp.float32),
        ],
    )(agg2, agg2, r0, g0, be0, wn, wr, b)


def _fuse_out_body(aggA, aggB, dg8, r1, g1, be1, batch, wc, bc, out,
                   p_acc, c_acc):
    i = pl.program_id(0)

    @pl.when(i == 0)
    def _():
        p_acc[...] = jnp.zeros_like(p_acc)
        c_acc[...] = jnp.zeros_like(c_acc)

    S = aggA[...] + aggB[...]
    h2 = _norm_layer(S, dg8[:, 0:1], r1, g1, be1)

    gids = lax.broadcasted_iota(jnp.int32, (BN, G), 1)
    m = (batch[...] == gids).astype(jnp.float32)
    p_acc[...] += lax.dot_general(m, h2, (((0,), (0,)), ((), ())),
                                  preferred_element_type=jnp.float32,
                                  precision=_HI)
    c_acc[...] += lax.dot_general(m, jnp.ones((BN, D), jnp.float32),
                                  (((0,), (0,)), ((), ())),
                                  preferred_element_type=jnp.float32,
                                  precision=_HI)

    @pl.when(i == NBLK - 1)
    def _():
        pooled = p_acc[...] / jnp.maximum(c_acc[...], 1.0)
        out[...] = jnp.dot(pooled, wc[...], preferred_element_type=jnp.float32,
                           precision=_HI) + bc[...]


def _fuse_out(agg2, dg8, r1, g1, be1, batch, wc, bc):
    return pl.pallas_call(
        _fuse_out_body,
        grid=(NBLK,),
        in_specs=[
            pl.BlockSpec((BN, D), lambda i: (i, 0)),
            pl.BlockSpec((BN, D), lambda i: (i + NBLK, 0)),
            pl.BlockSpec((BN, 8), lambda i: (i, 0)),
            pl.BlockSpec((BN, D), lambda i: (i, 0)),
            pl.BlockSpec((1, D), lambda i: (0, 0)),
            pl.BlockSpec((1, D), lambda i: (0, 0)),
            pl.BlockSpec((BN, 1), lambda i: (i, 0)),
            pl.BlockSpec((D, D), lambda i: (0, 0)),
            pl.BlockSpec((1, D), lambda i: (0, 0)),
        ],
        out_specs=pl.BlockSpec((G, D), lambda i: (0, 0)),
        out_shape=jax.ShapeDtypeStruct((G, D), jnp.float32),
        scratch_shapes=[
            pltpu.VMEM((G, D), jnp.float32),
            pltpu.VMEM((G, D), jnp.float32),
        ],
    )(agg2, agg2, dg8, r1, g1, be1, batch, wc, bc)


@jax.jit
def kernel(x, edge_index, batch, W_nbr0, W_root0, b0, g0, be0,
           W_nbr1, W_root1, b1, g1, be1, Wc, bc):
    f32 = jnp.float32
    x_pad = jnp.zeros((NP, D), f32).at[:N].set(x)
    pad_idx = DUMMY + jnp.arange(EP - E, dtype=jnp.int32) % (NP - N)
    src = jnp.concatenate([edge_index[0].astype(jnp.int32), pad_idx])
    src = src.reshape(EP // CH, CH)
    dst = jnp.concatenate([edge_index[1].astype(jnp.int32), pad_idx])
    dst = dst.reshape(EP // CH, CH)
    batch_pad = jnp.full((NP, 1), G, jnp.int32).at[:N, 0].set(batch)

    zrows0 = jnp.zeros((CH, W0), f32)
    zrows1 = jnp.zeros((CH, D), f32)

    b0r = b0.reshape(1, D)
    b1r = b1.reshape(1, D)
    g0r = g0.reshape(1, D)
    be0r = be0.reshape(1, D)
    g1r = g1.reshape(1, D)
    be1r = be1.reshape(1, D)
    bcr = bc.reshape(1, D)

    y0, r0 = _dense0(x_pad, W_nbr0, W_root0, b0r)
    agg0 = _segsum(y0, src, dst, zrows0)
    y1, r1, dg8 = _fuse_mid(agg0, r0, g0r, be0r, W_nbr1, W_root1, b1r)
    agg1 = _segsum(y1, src, dst, zrows1)
    out = _fuse_out(agg1, dg8, r1, g1r, be1r, batch_pad, Wc, bcr)
    return out


# speedup vs baseline: 3.2332x; 1.2930x over previous
import functools

import jax
import jax.numpy as jnp
from jax import lax
from jax.experimental import pallas as pl
from jax.experimental.pallas import tpu as pltpu
from jax.experimental.pallas import tpu_sc as plsc

N = 10000
D = 128
E = 320000
G = 32
W0 = D + 16

NP = 10112
ROWS_PER_TILE = NP // 16
NW = 32
CH = 128
MB = 8
NMACRO = -(-E // (NW * CH * MB))
NCHUNK = NMACRO * MB
EP = NW * CH * NCHUNK
PER_W = EP // NW
DUMMY = N


def _sc_segsum_body(width, table, src, dst, zrows, out, acc,
                    src_idx, dst_idx, rows0, rows1,
                    gsem0, gsem1, ssem0, ssem1):
    c = lax.axis_index("c")
    s = lax.axis_index("s")
    wid = s * 2 + c

    rows = (rows0, rows1)
    gsem = (gsem0, gsem1)
    ssem = (ssem0, ssem1)

    chunks = [(0, 128), (128, 128), (256, 128), (384, 128), (512, 120)]

    row0 = s * ROWS_PER_TILE
    pltpu.sync_copy(zrows, rows0)
    for off_r, cnt in chunks:
        pltpu.sync_copy(rows0.at[pl.ds(0, cnt)],
                        acc.at[pl.ds(row0 + off_r, cnt)])
    plsc.subcore_barrier()

    base = wid * NCHUNK

    def macro(t, _):
        crow = base + t * MB
        pltpu.sync_copy(src.at[pl.ds(crow, MB)], src_idx)
        pltpu.sync_copy(dst.at[pl.ds(crow, MB)], dst_idx)
        cps = [pltpu.async_copy(table.at[src_idx.at[0]], rows0, gsem0), None]
        for m in range(MB):
            if m + 1 < MB:
                cps[(m + 1) % 2] = pltpu.async_copy(
                    table.at[src_idx.at[m + 1]], rows[(m + 1) % 2],
                    gsem[(m + 1) % 2])
            cps[m % 2].wait()
            pltpu.sync_copy(rows[m % 2], acc.at[dst_idx.at[m]], add=True)
        return 0

    lax.fori_loop(0, NMACRO, macro, 0)
    plsc.subcore_barrier()

    out_row = c * NP + row0
    for off_r, cnt in chunks:
        pltpu.sync_copy(acc.at[pl.ds(row0 + off_r, cnt)],
                        rows0.at[pl.ds(0, cnt)])
        pltpu.sync_copy(rows0.at[pl.ds(0, cnt)],
                        out.at[pl.ds(out_row + off_r, cnt)])


@functools.lru_cache(maxsize=None)
def _sc_segsum_cached(width):
    mesh = plsc.VectorSubcoreMesh(core_axis_name="c", subcore_axis_name="s")
    return pl.kernel(
        functools.partial(_sc_segsum_body, width),
        out_type=jax.ShapeDtypeStruct((2 * NP, width), jnp.float32),
        mesh=mesh,
        scratch_types=[
            pltpu.VMEM_SHARED((NP, width), jnp.float32),
            pltpu.VMEM((MB, CH), jnp.int32),
            pltpu.VMEM((MB, CH), jnp.int32),
            pltpu.VMEM((CH, width), jnp.float32),
            pltpu.VMEM((CH, width), jnp.float32),
            pltpu.SemaphoreType.DMA,
            pltpu.SemaphoreType.DMA,
            pltpu.SemaphoreType.DMA,
            pltpu.SemaphoreType.DMA,
        ],
        compiler_params=pltpu.CompilerParams(use_tc_tiling_on_sc=False),
    )


def _segsum(table, src, dst, zrows):
    return _sc_segsum_cached(table.shape[1])(table, src, dst, zrows)


BN = 2528
NBLK = NP // BN
_HI = lax.Precision.HIGHEST


def _dense0_body(x, wn, wr, b, y, r):
    xb = x[...]
    y0 = jnp.dot(xb, wn[...], preferred_element_type=jnp.float32, precision=_HI)
    y[...] = jnp.concatenate([y0, jnp.ones((BN, 16), jnp.float32)], axis=1)
    r[...] = jnp.dot(xb, wr[...], preferred_element_type=jnp.float32,
                     precision=_HI) + b[...]


def _dense0(x, wn, wr, b):
    return pl.pallas_call(
        _dense0_body,
        grid=(NBLK,),
        in_specs=[
            pl.BlockSpec((BN, D), lambda i: (i, 0)),
            pl.BlockSpec((D, D), lambda i: (0, 0)),
            pl.BlockSpec((D, D), lambda i: (0, 0)),
            pl.BlockSpec((1, D), lambda i: (0, 0)),
        ],
        out_specs=[
            pl.BlockSpec((BN, W0), lambda i: (i, 0)),
            pl.BlockSpec((BN, D), lambda i: (i, 0)),
        ],
        out_shape=[
            jax.ShapeDtypeStruct((NP, W0), jnp.float32),
            jax.ShapeDtypeStruct((NP, D), jnp.float32),
        ],
    )(x, wn, wr, b)


def _norm_layer(msg, dg, r, g, be):
    h = msg / dg + r[...]
    mu = jnp.mean(h, axis=-1, keepdims=True)
    var = jnp.mean((h - mu) * (h - mu), axis=-1, keepdims=True)
    hn = (h - mu) * lax.rsqrt(var + 1e-5) * g[...] + be[...]
    return jnp.maximum(hn, 0.0)


def _fuse_mid_body(aggA, aggB, r0, g0, be0, wn, wr, b, y1, r1, dg8):
    S = aggA[...] + aggB[...]
    dg = jnp.maximum(S[:, D:D + 1], 1.0)
    h1 = _norm_layer(S[:, :D], dg, r0, g0, be0)
    y1[...] = jnp.dot(h1, wn[...], preferred_element_type=jnp.float32,
                      precision=_HI)
    r1[...] = jnp.dot(h1, wr[...], preferred_element_type=jnp.float32,
                      precision=_HI) + b[...]
    dg8[...] = lax.broadcast_in_dim(dg, (BN, 8), (0, 1))


def _fuse_mid(agg2, r0, g0, be0, wn, wr, b):
    return pl.pallas_call(
        _fuse_mid_body,
        grid=(NBLK,),
        in_specs=[
            pl.BlockSpec((BN, W0), lambda i: (i, 0)),
            pl.BlockSpec((BN, W0), lambda i: (i + NBLK, 0)),
            pl.BlockSpec((BN, D), lambda i: (i, 0)),
            pl.BlockSpec((1, D), lambda i: (0, 0)),
            pl.BlockSpec((1, D), lambda i: (0, 0)),
            pl.BlockSpec((D, D), lambda i: (0, 0)),
            pl.BlockSpec((D, D), lambda i: (0, 0)),
            pl.BlockSpec((1, D), lambda i: (0, 0)),
        ],
        out_specs=[
            pl.BlockSpec((BN, D), lambda i: (i, 0)),
            pl.BlockSpec((BN, D), lambda i: (i, 0)),
            pl.BlockSpec((BN, 8), lambda i: (i, 0)),
        ],
        out_shape=[
            jax.ShapeDtypeStruct((NP, D), jnp.float32),
            jax.ShapeDtypeStruct((NP, D), jnp.float32),
            jax.ShapeDtypeStruct((NP, 8), jnp.float32),
        ],
    )(agg2, agg2, r0, g0, be0, wn, wr, b)


def _fuse_out_body(aggA, aggB, dg8, r1, g1, be1, batch, wc, bc, out,
                   p_acc, c_acc):
    i = pl.program_id(0)

    @pl.when(i == 0)
    def _():
        p_acc[...] = jnp.zeros_like(p_acc)
        c_acc[...] = jnp.zeros_like(c_acc)

    S = aggA[...] + aggB[...]
    h2 = _norm_layer(S, dg8[:, 0:1], r1, g1, be1)

    gids = lax.broadcasted_iota(jnp.int32, (BN, G), 1)
    m = (batch[...] == gids).astype(jnp.float32)
    p_acc[...] += lax.dot_general(m, h2, (((0,), (0,)), ((), ())),
                                  preferred_element_type=jnp.float32,
                                  precision=_HI)
    c_acc[...] += lax.dot_general(m, jnp.ones((BN, D), jnp.float32),
                                  (((0,), (0,)), ((), ())),
                                  preferred_element_type=jnp.float32,
                                  precision=_HI)

    @pl.when(i == NBLK - 1)
    def _():
        pooled = p_acc[...] / jnp.maximum(c_acc[...], 1.0)
        out[...] = jnp.dot(pooled, wc[...], preferred_element_type=jnp.float32,
                           precision=_HI) + bc[...]


def _fuse_out(agg2, dg8, r1, g1, be1, batch, wc, bc):
    return pl.pallas_call(
        _fuse_out_body,
        grid=(NBLK,),
        in_specs=[
            pl.BlockSpec((BN, D), lambda i: (i, 0)),
            pl.BlockSpec((BN, D), lambda i: (i + NBLK, 0)),
            pl.BlockSpec((BN, 8), lambda i: (i, 0)),
            pl.BlockSpec((BN, D), lambda i: (i, 0)),
            pl.BlockSpec((1, D), lambda i: (0, 0)),
            pl.BlockSpec((1, D), lambda i: (0, 0)),
            pl.BlockSpec((BN, 1), lambda i: (i, 0)),
            pl.BlockSpec((D, D), lambda i: (0, 0)),
            pl.BlockSpec((1, D), lambda i: (0, 0)),
        ],
        out_specs=pl.BlockSpec((G, D), lambda i: (0, 0)),
        out_shape=jax.ShapeDtypeStruct((G, D), jnp.float32),
        scratch_shapes=[
            pltpu.VMEM((G, D), jnp.float32),
            pltpu.VMEM((G, D), jnp.float32),
        ],
    )(agg2, agg2, dg8, r1, g1, be1, batch, wc, bc)


@jax.jit
def kernel(x, edge_index, batch, W_nbr0, W_root0, b0, g0, be0,
           W_nbr1, W_root1, b1, g1, be1, Wc, bc):
    f32 = jnp.float32
    x_pad = jnp.zeros((NP, D), f32).at[:N].set(x)
    pad_idx = DUMMY + jnp.arange(EP - E, dtype=jnp.int32) % (NP - N)
    src = jnp.concatenate([edge_index[0].astype(jnp.int32), pad_idx])
    src = src.reshape(EP // CH, CH)
    dst = jnp.concatenate([edge_index[1].astype(jnp.int32), pad_idx])
    dst = dst.reshape(EP // CH, CH)
    batch_pad = jnp.full((NP, 1), G, jnp.int32).at[:N, 0].set(batch)

    zrows0 = jnp.zeros((CH, W0), f32)
    zrows1 = jnp.zeros((CH, D), f32)

    b0r = b0.reshape(1, D)
    b1r = b1.reshape(1, D)
    g0r = g0.reshape(1, D)
    be0r = be0.reshape(1, D)
    g1r = g1.reshape(1, D)
    be1r = be1.reshape(1, D)
    bcr = bc.reshape(1, D)

    y0, r0 = _dense0(x_pad, W_nbr0, W_root0, b0r)
    agg0 = _segsum(y0, src, dst, zrows0)
    y1, r1, dg8 = _fuse_mid(agg0, r0, g0r, be0r, W_nbr1, W_root1, b1r)
    agg1 = _segsum(y1, src, dst, zrows1)
    out = _fuse_out(agg1, dg8, r1, g1r, be1r, batch_pad, Wc, bcr)
    return out
